# Initial kernel scaffold; baseline (speedup 1.0000x reference)
import jax
import jax.numpy as jnp
from jax.experimental import pallas as pl


def kernel(x, W, b, vars_p, scaling):
    raise NotImplementedError("write your pallas kernel here")


# speedup vs baseline: 9.1770x; 9.1770x over previous
import functools

import jax
import jax.numpy as jnp
from jax import lax
from jax.experimental import pallas as pl
from jax.experimental.pallas import tpu as pltpu
from jax.experimental.pallas import tpu_sc as plsc

_B, _T, _C = 4, 2048, 1024
_G, _V = 2, 320
_VP = 384
_N = _B * _T
_NG = _N * _G
_VD = 256
_TB = 512
_NBLK = _N // _TB
_NEG = -1e30

_NC, _NS = 2, 16
_NW = _NC * _NS
_BPW = _NG // _NW
_CHUNKS = 4
_CB = _BPW // _CHUNKS


def _tc_body(x_ref, w_ref, b_ref, idx0_ref, idx1_ref,
             lent_ref, cperp_ref, pperp_ref,
             probs0, probs1, cnt0, cnt1, cs0, cs1):
    i = pl.program_id(0)

    @pl.when(i == 0)
    def _init():
        for r in (probs0, probs1, cnt0, cnt1, cs0, cs1):
            r[...] = jnp.zeros_like(r)

    xb = x_ref[...]
    lp = jnp.dot(xb, w_ref[...], preferred_element_type=jnp.float32)
    lp = lp + b_ref[...]

    for g, idx_ref, pa, ca, sa in ((0, idx0_ref, probs0, cnt0, cs0),
                                   (1, idx1_ref, probs1, cnt1, cs1)):
        lg = lp[:, g * _VP:(g + 1) * _VP]
        m = jnp.max(lg, axis=1, keepdims=True)
        e = jnp.exp(lg - m)
        p = e / jnp.sum(e, axis=1, keepdims=True)
        pa[...] += jnp.sum(p, axis=0, keepdims=True)
        k = jnp.argmax(lg, axis=1).astype(jnp.int32)
        idx_ref[...] = k
        oh = (lax.broadcasted_iota(jnp.int32, (_TB, _VP), 1)
              == k[:, None]).astype(jnp.float32)
        ca[...] += jnp.sum(oh, axis=0, keepdims=True)
        sa[...] += jnp.sum(lg, axis=0, keepdims=True)

    @pl.when(i == _NBLK - 1)
    def _finish():
        invn = jnp.float32(1.0 / _N)
        pperp = jnp.float32(0.0)
        cperp = jnp.float32(0.0)
        for pa, ca in ((probs0, cnt0), (probs1, cnt1)):
            ap = pa[...] * invn
            pperp += jnp.exp(-jnp.sum(ap * jnp.log(ap + 1e-7)))
            hp = ca[...] * invn
            cperp += jnp.exp(-jnp.sum(hp * jnp.log(hp + 1e-7)))
        x0 = cs0[...] * invn
        x1 = cs1[...] * invn
        m2 = jnp.maximum(jnp.max(x0), jnp.max(x1))
        e0 = jnp.exp(x0 - m2)
        e1 = jnp.exp(x1 - m2)
        z = jnp.sum(e0) + jnp.sum(e1)
        logz = jnp.log(z)
        lent = (jnp.sum(e0 * ((x0 - m2) - logz))
                + jnp.sum(e1 * ((x1 - m2) - logz))) / z
        lent_ref[0, 0] = lent
        cperp_ref[0, 0] = cperp
        pperp_ref[0, 0] = pperp


def _tc_stats(x2d, w_eff, b_eff):
    return pl.pallas_call(
        _tc_body,
        grid=(_NBLK,),
        in_specs=[
            pl.BlockSpec((_TB, _C), lambda i: (i, 0)),
            pl.BlockSpec((_C, _G * _VP), lambda i: (0, 0)),
            pl.BlockSpec((1, _G * _VP), lambda i: (0, 0)),
        ],
        out_specs=[
            pl.BlockSpec((_TB,), lambda i: (i,)),
            pl.BlockSpec((_TB,), lambda i: (i,)),
            pl.BlockSpec(memory_space=pltpu.SMEM),
            pl.BlockSpec(memory_space=pltpu.SMEM),
            pl.BlockSpec(memory_space=pltpu.SMEM),
        ],
        out_shape=[
            jax.ShapeDtypeStruct((_N,), jnp.int32),
            jax.ShapeDtypeStruct((_N,), jnp.int32),
            jax.ShapeDtypeStruct((1, 1), jnp.float32),
            jax.ShapeDtypeStruct((1, 1), jnp.float32),
            jax.ShapeDtypeStruct((1, 1), jnp.float32),
        ],
        scratch_shapes=[pltpu.VMEM((1, _VP), jnp.float32) for _ in range(6)],
        compiler_params=pltpu.CompilerParams(
            dimension_semantics=("arbitrary",)),
    )(x2d, w_eff, b_eff)


def _sc_body(table_hbm, idx_hbm, out_hbm, idx_v, buf0, buf1, sem0, sem1):
    wid = lax.axis_index("s") * _NC + lax.axis_index("c")
    pltpu.sync_copy(idx_hbm.at[pl.ds(wid * _CHUNKS, _CHUNKS)], idx_v)
    bufs = (buf0, buf1)
    sems = (sem0, sem1)
    handles = [None, None]
    handles[0] = pltpu.async_copy(table_hbm.at[idx_v.at[0]], buf0, sem0)
    for c in range(_CHUNKS):
        n = c + 1
        if n < _CHUNKS:
            handles[n % 2] = pltpu.async_copy(
                table_hbm.at[idx_v.at[n]], bufs[n % 2], sems[n % 2])
        handles[c % 2].wait()
        pltpu.sync_copy(bufs[c % 2],
                        out_hbm.at[pl.ds(wid * _BPW + c * _CB, _CB)])


def _sc_gather(table, idx2d):
    mesh = plsc.VectorSubcoreMesh(core_axis_name="c", subcore_axis_name="s")
    run = functools.partial(
        pl.kernel,
        mesh=mesh,
        out_type=jax.ShapeDtypeStruct((_NG, _VD), jnp.float32),
        scratch_types=[
            pltpu.VMEM((_CHUNKS, _CB), jnp.int32),
            pltpu.VMEM((_CB, _VD), jnp.float32),
            pltpu.VMEM((_CB, _VD), jnp.float32),
            pltpu.SemaphoreType.DMA,
            pltpu.SemaphoreType.DMA,
        ],
    )(_sc_body)
    return run(table, idx2d)


def kernel(x, W, b, vars_p, scaling):
    avg = scaling.mean()
    scale = 1.0 + 10.0 * (scaling - avg)
    wg = (W * scale[:, None]).reshape(_G, _V, _C)
    wp = jnp.pad(wg, ((0, 0), (0, _VP - _V), (0, 0)))
    w_eff = wp.transpose(2, 0, 1).reshape(_C, _G * _VP)
    bg = (b * scale).reshape(_G, _V)
    b_eff = jnp.pad(bg, ((0, 0), (0, _VP - _V)),
                    constant_values=_NEG).reshape(1, _G * _VP)

    x2d = x.reshape(_N, _C)
    k0, k1, lent, cperp, pperp = _tc_stats(x2d, w_eff, b_eff)

    idx = jnp.stack([k0, k1 + _V], axis=1).reshape(_NW * _CHUNKS, _CB)
    table = vars_p.reshape(_G * _V, _VD)
    rows = _sc_gather(table, idx)
    q = rows.reshape(_B, _T, _G * _VD)

    return (q, lent.reshape(()), cperp.reshape(()), pperp.reshape(()))


# speedup vs baseline: 9.6248x; 1.0488x over previous
import functools

import jax
import jax.numpy as jnp
from jax import lax
from jax.experimental import pallas as pl
from jax.experimental.pallas import tpu as pltpu
from jax.experimental.pallas import tpu_sc as plsc

_B, _T, _C = 4, 2048, 1024
_G, _V = 2, 320
_VP = 384
_NEG = -1e30
_N = _B * _T
_NG = _N * _G
_VD = 256
_TB = 512
_NBLK = _N // _TB
_TPW = 512

_NC, _NS = 2, 16
_NW = _NC * _NS
_CHUNKS = 4
_CB = _TPW // _CHUNKS


def _tc_body(x_ref, w_ref, sb_ref, s_ref, idx0_ref, idx1_ref,
             lent_ref, cperp_ref, pperp_ref,
             probs0, probs1, cnt0, cnt1, cs0, cs1):
    i = pl.program_id(0)

    @pl.when(i == 0)
    def _init():
        for r in (probs0, probs1, cnt0, cnt1, cs0, cs1):
            r[...] = jnp.zeros_like(r)

    xb = x_ref[...]
    rawp = lax.dot_general(xb, w_ref[...], (((1,), (1,)), ((), ())),
                           preferred_element_type=jnp.float32)
    for g, idx_ref, pa, ca, sa in (
            (0, idx0_ref, probs0, cnt0, cs0),
            (1, idx1_ref, probs1, cnt1, cs1)):
        raw = rawp[:, g * _VP:(g + 1) * _VP]
        lg = raw * s_ref[g, :][None, :] + sb_ref[g, :][None, :]
        m = jnp.max(lg, axis=1, keepdims=True)
        e = jnp.exp(lg - m)
        p = e / jnp.sum(e, axis=1, keepdims=True)
        pa[...] += jnp.sum(p, axis=0, keepdims=True)
        k = jnp.argmax(lg, axis=1).astype(jnp.int32)
        idx_ref[...] = k
        oh = (lax.broadcasted_iota(jnp.int32, (_TB, _VP), 1)
              == k[:, None]).astype(jnp.float32)
        ca[...] += jnp.sum(oh, axis=0, keepdims=True)
        sa[...] += jnp.sum(lg, axis=0, keepdims=True)

    @pl.when(i == _NBLK - 1)
    def _finish():
        invn = jnp.float32(1.0 / _N)
        pperp = jnp.float32(0.0)
        cperp = jnp.float32(0.0)
        for pa, ca in ((probs0, cnt0), (probs1, cnt1)):
            ap = pa[...] * invn
            pperp += jnp.exp(-jnp.sum(ap * jnp.log(ap + 1e-7)))
            hp = ca[...] * invn
            cperp += jnp.exp(-jnp.sum(hp * jnp.log(hp + 1e-7)))
        x0 = cs0[...] * invn
        x1 = cs1[...] * invn
        m2 = jnp.maximum(jnp.max(x0), jnp.max(x1))
        e0 = jnp.exp(x0 - m2)
        e1 = jnp.exp(x1 - m2)
        z = jnp.sum(e0) + jnp.sum(e1)
        logz = jnp.log(z)
        lent = (jnp.sum(e0 * ((x0 - m2) - logz))
                + jnp.sum(e1 * ((x1 - m2) - logz))) / z
        lent_ref[0, 0] = lent
        cperp_ref[0, 0] = cperp
        pperp_ref[0, 0] = pperp


def _tc_stats(x2d, wp, sb2, s2):
    return pl.pallas_call(
        _tc_body,
        grid=(_NBLK,),
        in_specs=[
            pl.BlockSpec((_TB, _C), lambda i: (i, 0)),
            pl.BlockSpec((_G * _VP, _C), lambda i: (0, 0)),
            pl.BlockSpec((_G, _VP), lambda i: (0, 0)),
            pl.BlockSpec((_G, _VP), lambda i: (0, 0)),
        ],
        out_specs=[
            pl.BlockSpec((_TB,), lambda i: (i,)),
            pl.BlockSpec((_TB,), lambda i: (i,)),
            pl.BlockSpec(memory_space=pltpu.SMEM),
            pl.BlockSpec(memory_space=pltpu.SMEM),
            pl.BlockSpec(memory_space=pltpu.SMEM),
        ],
        out_shape=[
            jax.ShapeDtypeStruct((_N,), jnp.int32),
            jax.ShapeDtypeStruct((_N,), jnp.int32),
            jax.ShapeDtypeStruct((1, 1), jnp.float32),
            jax.ShapeDtypeStruct((1, 1), jnp.float32),
            jax.ShapeDtypeStruct((1, 1), jnp.float32),
        ],
        scratch_shapes=[pltpu.VMEM((1, _VP), jnp.float32) for _ in range(6)],
        compiler_params=pltpu.CompilerParams(
            dimension_semantics=("arbitrary",)),
    )(x2d, wp, sb2, s2)


def _sc_body(table_hbm, kidx_hbm, oidx_hbm, out_hbm,
             kv, ov, buf0, buf1, gs0, gs1, ss0, ss1):
    wid = lax.axis_index("s") * _NC + lax.axis_index("c")
    g = wid & 1
    blk = wid >> 1
    row0 = g * (_NW // 2) * _CHUNKS + blk * _CHUNKS
    pltpu.sync_copy(kidx_hbm.at[pl.ds(row0, _CHUNKS)], kv)
    pltpu.sync_copy(oidx_hbm.at[pl.ds(row0, _CHUNKS)], ov)
    bufs = (buf0, buf1)
    gsems = (gs0, gs1)
    ssems = (ss0, ss1)
    gh = [None, None]
    sh = [None, None]
    gh[0] = pltpu.async_copy(table_hbm.at[kv.at[0]], buf0, gs0)
    for c in range(_CHUNKS):
        n = c + 1
        if n < _CHUNKS:
            if sh[n % 2] is not None:
                sh[n % 2].wait()
            gh[n % 2] = pltpu.async_copy(
                table_hbm.at[kv.at[n]], bufs[n % 2], gsems[n % 2])
        gh[c % 2].wait()
        sh[c % 2] = pltpu.async_copy(
            bufs[c % 2], out_hbm.at[ov.at[c]], ssems[c % 2])
    sh[(_CHUNKS - 2) % 2].wait()
    sh[(_CHUNKS - 1) % 2].wait()


def _sc_gather(table, kidx, oidx):
    mesh = plsc.VectorSubcoreMesh(core_axis_name="c", subcore_axis_name="s")
    run = functools.partial(
        pl.kernel,
        mesh=mesh,
        out_type=jax.ShapeDtypeStruct((_NG, _VD), jnp.float32),
        scratch_types=[
            pltpu.VMEM((_CHUNKS, _CB), jnp.int32),
            pltpu.VMEM((_CHUNKS, _CB), jnp.int32),
            pltpu.VMEM((_CB, _VD), jnp.float32),
            pltpu.VMEM((_CB, _VD), jnp.float32),
            pltpu.SemaphoreType.DMA,
            pltpu.SemaphoreType.DMA,
            pltpu.SemaphoreType.DMA,
            pltpu.SemaphoreType.DMA,
        ],
    )(_sc_body)
    return run(table, kidx, oidx)


def kernel(x, W, b, vars_p, scaling):
    avg = scaling.mean()
    scale = 1.0 + 10.0 * (scaling - avg)
    s2 = jnp.pad(scale.reshape(_G, _V), ((0, 0), (0, _VP - _V)),
                 constant_values=1.0)
    sb2 = jnp.pad((b * scale).reshape(_G, _V), ((0, 0), (0, _VP - _V)),
                  constant_values=_NEG)
    wp = jnp.pad(W.reshape(_G, _V, _C),
                 ((0, 0), (0, _VP - _V), (0, 0))).reshape(_G * _VP, _C)

    x2d = x.reshape(_N, _C)
    k0, k1, lent, cperp, pperp = _tc_stats(x2d, wp, sb2, s2)

    kidx = jnp.concatenate([k0, k1 + _V]).reshape(_NW * _CHUNKS, _CB)
    j = jnp.arange(_TPW, dtype=jnp.int32).reshape(_CHUNKS, _CB)
    blocks = jnp.arange(_NW // 2, dtype=jnp.int32) * _TPW
    rows = 2 * (blocks[:, None, None] + j[None])
    oidx = jnp.stack([rows, rows + 1]).reshape(_NW * _CHUNKS, _CB)

    table = vars_p.reshape(_G * _V, _VD)
    out = _sc_gather(table, kidx, oidx)
    q = out.reshape(_B, _T, _G * _VD)

    return (q, lent.reshape(()), cperp.reshape(()), pperp.reshape(()))


# speedup vs baseline: 11.7090x; 1.2166x over previous
import functools

import jax
import jax.numpy as jnp
from jax import lax
from jax.experimental import pallas as pl
from jax.experimental.pallas import tpu as pltpu
from jax.experimental.pallas import tpu_sc as plsc

_B, _T, _C = 4, 2048, 1024
_G, _V = 2, 320
_VP = 384
_NEG = -1e30
_N = _B * _T
_NG = _N * _G
_VD = 256
_TB = 512
_NBLK = _N // _TB
_TPW = 512

_NC, _NS = 2, 16
_NW = _NC * _NS
_CHUNKS = 4
_CB = _TPW // _CHUNKS


def _tc_body(x_ref, w_ref, sb_ref, s_ref, idx0_ref, idx1_ref,
             lent_ref, cperp_ref, pperp_ref,
             probs0, probs1, cnt0, cnt1, cs0, cs1):
    i = pl.program_id(0)

    @pl.when(i == 0)
    def _init():
        for r in (probs0, probs1, cnt0, cnt1, cs0, cs1):
            r[...] = jnp.zeros_like(r)

    xb = x_ref[...]
    rawp = lax.dot_general(xb, w_ref[...], (((1,), (1,)), ((), ())),
                           preferred_element_type=jnp.float32)
    for g, idx_ref, pa, ca, sa in (
            (0, idx0_ref, probs0, cnt0, cs0),
            (1, idx1_ref, probs1, cnt1, cs1)):
        raw = rawp[:, g * _VP:(g + 1) * _VP]
        lg = raw * s_ref[g, :][None, :] + sb_ref[g, :][None, :]
        m = jnp.max(lg, axis=1, keepdims=True)
        e = jnp.exp(lg - m)
        p = e / jnp.sum(e, axis=1, keepdims=True)
        pa[...] += jnp.sum(p, axis=0, keepdims=True)
        k = jnp.argmax(lg, axis=1).astype(jnp.int32)
        idx_ref[...] = k
        oh = (lax.broadcasted_iota(jnp.int32, (_TB, _VP), 1)
              == k[:, None]).astype(jnp.float32)
        ca[...] += jnp.sum(oh, axis=0, keepdims=True)
        sa[...] += jnp.sum(lg, axis=0, keepdims=True)

    @pl.when(i == _NBLK - 1)
    def _finish():
        invn = jnp.float32(1.0 / _N)
        pperp = jnp.float32(0.0)
        cperp = jnp.float32(0.0)
        for pa, ca in ((probs0, cnt0), (probs1, cnt1)):
            ap = pa[...] * invn
            pperp += jnp.exp(-jnp.sum(ap * jnp.log(ap + 1e-7)))
            hp = ca[...] * invn
            cperp += jnp.exp(-jnp.sum(hp * jnp.log(hp + 1e-7)))
        x0 = cs0[...] * invn
        x1 = cs1[...] * invn
        m2 = jnp.maximum(jnp.max(x0), jnp.max(x1))
        e0 = jnp.exp(x0 - m2)
        e1 = jnp.exp(x1 - m2)
        z = jnp.sum(e0) + jnp.sum(e1)
        logz = jnp.log(z)
        lent = (jnp.sum(e0 * ((x0 - m2) - logz))
                + jnp.sum(e1 * ((x1 - m2) - logz))) / z
        lent_ref[0, 0] = lent
        cperp_ref[0, 0] = cperp
        pperp_ref[0, 0] = pperp


def _tc_stats(x2d, wp, sb2, s2):
    return pl.pallas_call(
        _tc_body,
        grid=(_NBLK,),
        in_specs=[
            pl.BlockSpec((_TB, _C), lambda i: (i, 0)),
            pl.BlockSpec((_G * _VP, _C), lambda i: (0, 0)),
            pl.BlockSpec((_G, _VP), lambda i: (0, 0)),
            pl.BlockSpec((_G, _VP), lambda i: (0, 0)),
        ],
        out_specs=[
            pl.BlockSpec((_TB,), lambda i: (i,)),
            pl.BlockSpec((_TB,), lambda i: (i,)),
            pl.BlockSpec(memory_space=pltpu.SMEM),
            pl.BlockSpec(memory_space=pltpu.SMEM),
            pl.BlockSpec(memory_space=pltpu.SMEM),
        ],
        out_shape=[
            jax.ShapeDtypeStruct((_N,), jnp.int32),
            jax.ShapeDtypeStruct((_N,), jnp.int32),
            jax.ShapeDtypeStruct((1, 1), jnp.float32),
            jax.ShapeDtypeStruct((1, 1), jnp.float32),
            jax.ShapeDtypeStruct((1, 1), jnp.float32),
        ],
        scratch_shapes=[pltpu.VMEM((1, _VP), jnp.float32) for _ in range(6)],
        compiler_params=pltpu.CompilerParams(
            dimension_semantics=("arbitrary",)),
    )(x2d, wp, sb2, s2)


def _sc_body(table_hbm, kidx_hbm, out_hbm,
             kv, buf0, buf1, gs0, gs1, ss0, ss1):
    wid = lax.axis_index("s") * _NC + lax.axis_index("c")
    g = wid & 1
    blk = wid >> 1
    row0 = g * (_NW // 2) * _CHUNKS + blk * _CHUNKS
    col0 = g * _VD
    tok0 = blk * _TPW
    pltpu.sync_copy(kidx_hbm.at[pl.ds(row0, _CHUNKS)], kv)
    bufs = (buf0, buf1)
    gsems = (gs0, gs1)
    ssems = (ss0, ss1)
    gh = [None, None]
    sh = [None, None]
    gh[0] = pltpu.async_copy(table_hbm.at[kv.at[0]], buf0, gs0)
    for c in range(_CHUNKS):
        n = c + 1
        if n < _CHUNKS:
            if sh[n % 2] is not None:
                sh[n % 2].wait()
            gh[n % 2] = pltpu.async_copy(
                table_hbm.at[kv.at[n]], bufs[n % 2], gsems[n % 2])
        gh[c % 2].wait()
        sh[c % 2] = pltpu.async_copy(
            bufs[c % 2],
            out_hbm.at[pl.ds(tok0 + c * _CB, _CB), pl.ds(col0, _VD)],
            ssems[c % 2])
    sh[(_CHUNKS - 2) % 2].wait()
    sh[(_CHUNKS - 1) % 2].wait()


def _sc_gather(table, kidx):
    mesh = plsc.VectorSubcoreMesh(core_axis_name="c", subcore_axis_name="s")
    run = functools.partial(
        pl.kernel,
        mesh=mesh,
        out_type=jax.ShapeDtypeStruct((_N, _G * _VD), jnp.float32),
        scratch_types=[
            pltpu.VMEM((_CHUNKS, _CB), jnp.int32),
            pltpu.VMEM((_CB, _VD), jnp.float32),
            pltpu.VMEM((_CB, _VD), jnp.float32),
            pltpu.SemaphoreType.DMA,
            pltpu.SemaphoreType.DMA,
            pltpu.SemaphoreType.DMA,
            pltpu.SemaphoreType.DMA,
        ],
    )(_sc_body)
    return run(table, kidx)


def kernel(x, W, b, vars_p, scaling):
    avg = scaling.mean()
    scale = 1.0 + 10.0 * (scaling - avg)
    s2 = jnp.pad(scale.reshape(_G, _V), ((0, 0), (0, _VP - _V)),
                 constant_values=1.0)
    sb2 = jnp.pad((b * scale).reshape(_G, _V), ((0, 0), (0, _VP - _V)),
                  constant_values=_NEG)
    wp = jnp.pad(W.reshape(_G, _V, _C),
                 ((0, 0), (0, _VP - _V), (0, 0))).reshape(_G * _VP, _C)

    x2d = x.reshape(_N, _C)
    k0, k1, lent, cperp, pperp = _tc_stats(x2d, wp, sb2, s2)

    kidx = jnp.concatenate([k0, k1 + _V]).reshape(_NW * _CHUNKS, _CB)

    table = vars_p.reshape(_G * _V, _VD)
    out = _sc_gather(table, kidx)
    q = out.reshape(_B, _T, _G * _VD)

    return (q, lent.reshape(()), cperp.reshape(()), pperp.reshape(()))


# speedup vs baseline: 11.9016x; 1.0164x over previous
import functools

import jax
import jax.numpy as jnp
from jax import lax
from jax.experimental import pallas as pl
from jax.experimental.pallas import tpu as pltpu
from jax.experimental.pallas import tpu_sc as plsc

_B, _T, _C = 4, 2048, 1024
_G, _V = 2, 320
_VP = 384
_NEG = -1e30
_N = _B * _T
_VD = 256
_TB = 512
_NSLICE = 2
_NSL = _N // _NSLICE
_NBLK = _NSL // _TB

_NC, _NS = 2, 16
_NW = _NC * _NS
_TPW = _NSL // (_NW // 2)
_CB = 128
_CHUNKS = _TPW // _CB


def _tc_body(last, x_ref, w_ref, sb_ref, s_ref, a0, a1, a2, a3, a4, a5,
             *out_refs):
    if last:
        (idx0_ref, idx1_ref, o0, o1, o2, o3, o4, o5,
         lent_ref, cperp_ref, pperp_ref,
         probs0, probs1, cnt0, cnt1, cs0, cs1) = out_refs
    else:
        (idx0_ref, idx1_ref, o0, o1, o2, o3, o4, o5,
         probs0, probs1, cnt0, cnt1, cs0, cs1) = out_refs
    acc_in = (a0, a1, a2, a3, a4, a5)
    acc_out = (o0, o1, o2, o3, o4, o5)
    scratch = (probs0, probs1, cnt0, cnt1, cs0, cs1)
    i = pl.program_id(0)

    @pl.when(i == 0)
    def _init():
        for r, src in zip(scratch, acc_in):
            r[...] = src[...]

    xb = x_ref[...]
    rawp = lax.dot_general(xb, w_ref[...], (((1,), (1,)), ((), ())),
                           preferred_element_type=jnp.float32)
    for g, idx_ref, pa, ca, sa in (
            (0, idx0_ref, probs0, cnt0, cs0),
            (1, idx1_ref, probs1, cnt1, cs1)):
        raw = rawp[:, g * _VP:(g + 1) * _VP]
        lg = raw * s_ref[g, :][None, :] + sb_ref[g, :][None, :]
        m = jnp.max(lg, axis=1, keepdims=True)
        e = jnp.exp(lg - m)
        p = e / jnp.sum(e, axis=1, keepdims=True)
        pa[...] += jnp.sum(p, axis=0, keepdims=True)
        k = jnp.argmax(lg, axis=1).astype(jnp.int32)
        idx_ref[...] = k
        oh = (lax.broadcasted_iota(jnp.int32, (_TB, _VP), 1)
              == k[:, None]).astype(jnp.float32)
        ca[...] += jnp.sum(oh, axis=0, keepdims=True)
        sa[...] += jnp.sum(lg, axis=0, keepdims=True)

    @pl.when(i == _NBLK - 1)
    def _emit():
        for r, dst in zip(scratch, acc_out):
            dst[...] = r[...]

    if last:
        @pl.when(i == _NBLK - 1)
        def _finish():
            invn = jnp.float32(1.0 / _N)
            pperp = jnp.float32(0.0)
            cperp = jnp.float32(0.0)
            for pa, ca in ((probs0, cnt0), (probs1, cnt1)):
                ap = pa[...] * invn
                pperp += jnp.exp(-jnp.sum(ap * jnp.log(ap + 1e-7)))
                hp = ca[...] * invn
                cperp += jnp.exp(-jnp.sum(hp * jnp.log(hp + 1e-7)))
            x0 = cs0[...] * invn
            x1 = cs1[...] * invn
            m2 = jnp.maximum(jnp.max(x0), jnp.max(x1))
            e0 = jnp.exp(x0 - m2)
            e1 = jnp.exp(x1 - m2)
            z = jnp.sum(e0) + jnp.sum(e1)
            logz = jnp.log(z)
            lent = (jnp.sum(e0 * ((x0 - m2) - logz))
                    + jnp.sum(e1 * ((x1 - m2) - logz))) / z
            lent_ref[0, 0] = lent
            cperp_ref[0, 0] = cperp
            pperp_ref[0, 0] = pperp


def _tc_stats(x2d, wp, sb2, s2, accs, base, last):
    acc_sds = jax.ShapeDtypeStruct((1, _VP), jnp.float32)
    out_shape = [jax.ShapeDtypeStruct((_NSL,), jnp.int32)] * 2 + [acc_sds] * 6
    out_specs = [pl.BlockSpec((_TB,), lambda i: (i,))] * 2 + \
                [pl.BlockSpec((1, _VP), lambda i: (0, 0))] * 6
    if last:
        out_shape += [jax.ShapeDtypeStruct((1, 1), jnp.float32)] * 3
        out_specs += [pl.BlockSpec(memory_space=pltpu.SMEM)] * 3
    return pl.pallas_call(
        functools.partial(_tc_body, last),
        grid=(_NBLK,),
        in_specs=[
            pl.BlockSpec((_TB, _C), lambda i: (i + base, 0)),
            pl.BlockSpec((_G * _VP, _C), lambda i: (0, 0)),
            pl.BlockSpec((_G, _VP), lambda i: (0, 0)),
            pl.BlockSpec((_G, _VP), lambda i: (0, 0)),
        ] + [pl.BlockSpec((1, _VP), lambda i: (0, 0))] * 6,
        out_specs=out_specs,
        out_shape=out_shape,
        scratch_shapes=[pltpu.VMEM((1, _VP), jnp.float32) for _ in range(6)],
        compiler_params=pltpu.CompilerParams(
            dimension_semantics=("arbitrary",)),
    )(x2d, wp, sb2, s2, *accs)


def _sc_body(base_tok, table_hbm, kidx_hbm, out_hbm,
             kv, buf0, buf1, gs0, gs1, ss0, ss1):
    wid = lax.axis_index("s") * _NC + lax.axis_index("c")
    g = wid & 1
    blk = wid >> 1
    row0 = g * (_NW // 2) * _CHUNKS + blk * _CHUNKS
    col0 = g * _VD
    tok0 = base_tok + blk * _TPW
    pltpu.sync_copy(kidx_hbm.at[pl.ds(row0, _CHUNKS)], kv)
    bufs = (buf0, buf1)
    gsems = (gs0, gs1)
    ssems = (ss0, ss1)
    gh = [None, None]
    sh = [None, None]
    gh[0] = pltpu.async_copy(table_hbm.at[kv.at[0]], buf0, gs0)
    for c in range(_CHUNKS):
        n = c + 1
        if n < _CHUNKS:
            if sh[n % 2] is not None:
                sh[n % 2].wait()
            gh[n % 2] = pltpu.async_copy(
                table_hbm.at[kv.at[n]], bufs[n % 2], gsems[n % 2])
        gh[c % 2].wait()
        sh[c % 2] = pltpu.async_copy(
            bufs[c % 2],
            out_hbm.at[pl.ds(tok0 + c * _CB, _CB), pl.ds(col0, _VD)],
            ssems[c % 2])
    for c in range(max(0, _CHUNKS - 2), _CHUNKS):
        sh[c % 2].wait()


def _sc_gather(table, kidx, qref, base_tok):
    mesh = plsc.VectorSubcoreMesh(core_axis_name="c", subcore_axis_name="s")
    out_type = () if qref is not None else jax.ShapeDtypeStruct(
        (_N, _G * _VD), jnp.float32)
    run = functools.partial(
        pl.kernel,
        mesh=mesh,
        out_type=out_type,
        scratch_types=[
            pltpu.VMEM((_CHUNKS, _CB), jnp.int32),
            pltpu.VMEM((_CB, _VD), jnp.float32),
            pltpu.VMEM((_CB, _VD), jnp.float32),
            pltpu.SemaphoreType.DMA,
            pltpu.SemaphoreType.DMA,
            pltpu.SemaphoreType.DMA,
            pltpu.SemaphoreType.DMA,
        ],
    )(functools.partial(_sc_body, base_tok))
    if qref is None:
        return run(table, kidx)
    run(table, kidx, qref)


def kernel(x, W, b, vars_p, scaling):
    avg = scaling.mean()
    scale = 1.0 + 10.0 * (scaling - avg)
    s2 = jnp.pad(scale.reshape(_G, _V), ((0, 0), (0, _VP - _V)),
                 constant_values=1.0)
    sb2 = jnp.pad((b * scale).reshape(_G, _V), ((0, 0), (0, _VP - _V)),
                  constant_values=_NEG)
    wp = jnp.pad(W.reshape(_G, _V, _C),
                 ((0, 0), (0, _VP - _V), (0, 0))).reshape(_G * _VP, _C)

    x2d = x.reshape(_N, _C)
    table = vars_p.reshape(_G * _V, _VD)
    zacc = [jnp.zeros((1, _VP), jnp.float32)] * 6

    k0a, k1a, *accs_a = _tc_stats(x2d, wp, sb2, s2, zacc, 0, False)
    kidx_a = jnp.concatenate([k0a, k1a + _V]).reshape(-1, _CB)
    out_a = _sc_gather(table, kidx_a, None, 0)
    qref = jax.new_ref(out_a)

    out_b = _tc_stats(x2d, wp, sb2, s2, accs_a, _NBLK, True)
    k0b, k1b = out_b[0], out_b[1]
    lent, cperp, pperp = out_b[8], out_b[9], out_b[10]
    kidx_b = jnp.concatenate([k0b, k1b + _V]).reshape(-1, _CB)
    _sc_gather(table, kidx_b, qref, _NSL)

    q = qref[...].reshape(_B, _T, _G * _VD)
    return (q, lent.reshape(()), cperp.reshape(()), pperp.reshape(()))


# speedup vs baseline: 12.4528x; 1.0463x over previous
import functools

import jax
import jax.numpy as jnp
from jax import lax
from jax.experimental import pallas as pl
from jax.experimental.pallas import tpu as pltpu
from jax.experimental.pallas import tpu_sc as plsc

_B, _T, _C = 4, 2048, 1024
_G, _V = 2, 320
_VP = 384
_NEG = -1e30
_N = _B * _T
_VD = 256
_TB = 512
_NSLICE = 2
_NSL = _N // _NSLICE
_NBLK = _NSL // _TB

_NC, _NS = 2, 16
_NW = _NC * _NS
_TPW = _NSL // (_NW // 2)
_CB = 128
_CHUNKS = _TPW // _CB


def _tc_body(last, x_ref, w_ref, sb_ref, s_ref, a0, a1, a2, a3, a4, a5,
             *out_refs):
    if last:
        (idx0_ref, idx1_ref, o0, o1, o2, o3, o4, o5,
         lent_ref, cperp_ref, pperp_ref,
         probs0, probs1, cnt0, cnt1, cs0, cs1) = out_refs
    else:
        (idx0_ref, idx1_ref, o0, o1, o2, o3, o4, o5,
         probs0, probs1, cnt0, cnt1, cs0, cs1) = out_refs
    acc_in = (a0, a1, a2, a3, a4, a5)
    acc_out = (o0, o1, o2, o3, o4, o5)
    scratch = (probs0, probs1, cnt0, cnt1, cs0, cs1)
    i = pl.program_id(0)

    @pl.when(i == 0)
    def _init():
        for r, src in zip(scratch, acc_in):
            r[...] = src[...]

    xb = x_ref[...]
    rawp = lax.dot_general(xb, w_ref[...], (((1,), (1,)), ((), ())),
                           preferred_element_type=jnp.float32)
    for g, idx_ref, pa, ca, sa in (
            (0, idx0_ref, probs0, cnt0, cs0),
            (1, idx1_ref, probs1, cnt1, cs1)):
        raw = rawp[:, g * _VP:(g + 1) * _VP]
        lg = raw * s_ref[g, :][None, :] + sb_ref[g, :][None, :]
        m = jnp.max(lg, axis=1, keepdims=True)
        e = jnp.exp(lg - m)
        p = e / jnp.sum(e, axis=1, keepdims=True)
        pa[...] += jnp.sum(p, axis=0, keepdims=True)
        k = jnp.argmax(lg, axis=1).astype(jnp.int32)
        idx_ref[...] = k
        oh = (lax.broadcasted_iota(jnp.int32, (_TB, _VP), 1)
              == k[:, None]).astype(jnp.float32)
        ca[...] += jnp.sum(oh, axis=0, keepdims=True)
        sa[...] += jnp.sum(lg, axis=0, keepdims=True)

    @pl.when(i == _NBLK - 1)
    def _emit():
        for r, dst in zip(scratch, acc_out):
            dst[...] = r[...]

    if last:
        @pl.when(i == _NBLK - 1)
        def _finish():
            invn = jnp.float32(1.0 / _N)
            pperp = jnp.float32(0.0)
            cperp = jnp.float32(0.0)
            for pa, ca in ((probs0, cnt0), (probs1, cnt1)):
                ap = pa[...] * invn
                pperp += jnp.exp(-jnp.sum(ap * jnp.log(ap + 1e-7)))
                hp = ca[...] * invn
                cperp += jnp.exp(-jnp.sum(hp * jnp.log(hp + 1e-7)))
            x0 = cs0[...] * invn
            x1 = cs1[...] * invn
            m2 = jnp.maximum(jnp.max(x0), jnp.max(x1))
            e0 = jnp.exp(x0 - m2)
            e1 = jnp.exp(x1 - m2)
            z = jnp.sum(e0) + jnp.sum(e1)
            logz = jnp.log(z)
            lent = (jnp.sum(e0 * ((x0 - m2) - logz))
                    + jnp.sum(e1 * ((x1 - m2) - logz))) / z
            lent_ref[0, 0] = lent
            cperp_ref[0, 0] = cperp
            pperp_ref[0, 0] = pperp


def _tc_stats(x2d, wp, sb2, s2, accs, base, last):
    acc_sds = jax.ShapeDtypeStruct((1, _VP), jnp.float32)
    out_shape = [jax.ShapeDtypeStruct((_NSL,), jnp.int32)] * 2 + [acc_sds] * 6
    out_specs = [pl.BlockSpec((_TB,), lambda i: (i,))] * 2 + \
                [pl.BlockSpec((1, _VP), lambda i: (0, 0))] * 6
    if last:
        out_shape += [jax.ShapeDtypeStruct((1, 1), jnp.float32)] * 3
        out_specs += [pl.BlockSpec(memory_space=pltpu.SMEM)] * 3
    return pl.pallas_call(
        functools.partial(_tc_body, last),
        grid=(_NBLK,),
        in_specs=[
            pl.BlockSpec((_TB, _C), lambda i: (i + base, 0)),
            pl.BlockSpec((_G * _VP, _C), lambda i: (0, 0)),
            pl.BlockSpec((_G, _VP), lambda i: (0, 0)),
            pl.BlockSpec((_G, _VP), lambda i: (0, 0)),
        ] + [pl.BlockSpec((1, _VP), lambda i: (0, 0))] * 6,
        out_specs=out_specs,
        out_shape=out_shape,
        scratch_shapes=[pltpu.VMEM((1, _VP), jnp.float32) for _ in range(6)],
        compiler_params=pltpu.CompilerParams(
            dimension_semantics=("arbitrary",)),
    )(x2d, wp, sb2, s2, *accs)


def _sc_body(base_tok, tableg_hbm, k0_hbm, k1_hbm, out_hbm,
             kv, buf0, buf1, gs0, gs1, ss0, ss1):
    wid = lax.axis_index("s") * _NC + lax.axis_index("c")
    g = wid & 1
    blk = wid >> 1
    col0 = g * _VD
    tok0 = base_tok + blk * _TPW

    @pl.when(g == 0)
    def _load0():
        pltpu.sync_copy(k0_hbm.at[pl.ds(blk * _TPW, _TPW)], kv)

    @pl.when(g == 1)
    def _load1():
        pltpu.sync_copy(k1_hbm.at[pl.ds(blk * _TPW, _TPW)], kv)

    tab = tableg_hbm.at[g]
    bufs = (buf0, buf1)
    gsems = (gs0, gs1)
    ssems = (ss0, ss1)
    gh = [None, None]
    sh = [None, None]
    gh[0] = pltpu.async_copy(tab.at[kv.at[pl.ds(0, _CB)]], buf0, gs0)
    for c in range(_CHUNKS):
        n = c + 1
        if n < _CHUNKS:
            if sh[n % 2] is not None:
                sh[n % 2].wait()
            gh[n % 2] = pltpu.async_copy(
                tab.at[kv.at[pl.ds(n * _CB, _CB)]], bufs[n % 2],
                gsems[n % 2])
        gh[c % 2].wait()
        sh[c % 2] = pltpu.async_copy(
            bufs[c % 2],
            out_hbm.at[pl.ds(tok0 + c * _CB, _CB), pl.ds(col0, _VD)],
            ssems[c % 2])
    for c in range(max(0, _CHUNKS - 2), _CHUNKS):
        sh[c % 2].wait()


def _sc_gather(tableg, k0, k1, qref, base_tok):
    mesh = plsc.VectorSubcoreMesh(core_axis_name="c", subcore_axis_name="s")
    out_type = () if qref is not None else jax.ShapeDtypeStruct(
        (_N, _G * _VD), jnp.float32)
    run = functools.partial(
        pl.kernel,
        mesh=mesh,
        out_type=out_type,
        scratch_types=[
            pltpu.VMEM((_TPW,), jnp.int32),
            pltpu.VMEM((_CB, _VD), jnp.float32),
            pltpu.VMEM((_CB, _VD), jnp.float32),
            pltpu.SemaphoreType.DMA,
            pltpu.SemaphoreType.DMA,
            pltpu.SemaphoreType.DMA,
            pltpu.SemaphoreType.DMA,
        ],
    )(functools.partial(_sc_body, base_tok))
    if qref is None:
        return run(tableg, k0, k1)
    run(tableg, k0, k1, qref)


def kernel(x, W, b, vars_p, scaling):
    avg = scaling.mean()
    scale = 1.0 + 10.0 * (scaling - avg)
    s2 = jnp.pad(scale.reshape(_G, _V), ((0, 0), (0, _VP - _V)),
                 constant_values=1.0)
    sb2 = jnp.pad((b * scale).reshape(_G, _V), ((0, 0), (0, _VP - _V)),
                  constant_values=_NEG)
    wp = jnp.pad(W.reshape(_G, _V, _C),
                 ((0, 0), (0, _VP - _V), (0, 0))).reshape(_G * _VP, _C)

    x2d = x.reshape(_N, _C)
    tableg = vars_p.reshape(_G, _V, _VD)
    zacc = [jnp.zeros((1, _VP), jnp.float32)] * 6

    k0a, k1a, *accs_a = _tc_stats(x2d, wp, sb2, s2, zacc, 0, False)
    out_a = _sc_gather(tableg, k0a, k1a, None, 0)
    qref = jax.new_ref(out_a)

    out_b = _tc_stats(x2d, wp, sb2, s2, accs_a, _NBLK, True)
    k0b, k1b = out_b[0], out_b[1]
    lent, cperp, pperp = out_b[8], out_b[9], out_b[10]
    _sc_gather(tableg, k0b, k1b, qref, _NSL)

    q = qref[...].reshape(_B, _T, _G * _VD)
    return (q, lent.reshape(()), cperp.reshape(()), pperp.reshape(()))


# speedup vs baseline: 13.5613x; 1.0890x over previous
import functools

import jax
import jax.numpy as jnp
from jax import lax
from jax.experimental import pallas as pl
from jax.experimental.pallas import tpu as pltpu
from jax.experimental.pallas import tpu_sc as plsc

_B, _T, _C = 4, 2048, 1024
_G, _V = 2, 320
_VP = 384
_NEG = -1e30
_N = _B * _T
_VD = 256
_TB = 512
_NBLK_A = 12
_NBLK_B = _N // _TB - _NBLK_A
_NSL_A = _NBLK_A * _TB
_NSL_B = _NBLK_B * _TB

_NC, _NS = 2, 16
_NW = _NC * _NS
_CB = 128


def _tc_body(nblk, last, *refs):
    if last:
        (x_ref, w_ref, sb_ref, s_ref, a0, a1, a2, a3, a4, a5,
         idx0_ref, idx1_ref, o0, o1, o2, o3, o4, o5,
         lent_ref, cperp_ref, pperp_ref,
         wsc, probs0, probs1, cnt0, cnt1, cs0, cs1) = refs
        acc_in = (a0, a1, a2, a3, a4, a5)
    else:
        (x_ref, w_ref, sb_ref, s_ref,
         idx0_ref, idx1_ref, o0, o1, o2, o3, o4, o5,
         wsc, probs0, probs1, cnt0, cnt1, cs0, cs1) = refs
        acc_in = None
    acc_out = (o0, o1, o2, o3, o4, o5)
    scratch = (probs0, probs1, cnt0, cnt1, cs0, cs1)
    i = pl.program_id(0)

    @pl.when(i == 0)
    def _init():
        if acc_in is None:
            for r in scratch:
                r[...] = jnp.zeros_like(r)
        else:
            for r, src in zip(scratch, acc_in):
                r[...] = src[...]
        wsc[0:_V, :] = w_ref[0:_V, :]
        wsc[_V:_VP, :] = jnp.zeros((_VP - _V, _C), jnp.float32)
        wsc[_VP:_VP + _V, :] = w_ref[_V:2 * _V, :]
        wsc[_VP + _V:2 * _VP, :] = jnp.zeros((_VP - _V, _C), jnp.float32)

    xb = x_ref[...]
    rawp = lax.dot_general(xb, wsc[...], (((1,), (1,)), ((), ())),
                           preferred_element_type=jnp.float32)
    for g, idx_ref, pa, ca, sa in (
            (0, idx0_ref, probs0, cnt0, cs0),
            (1, idx1_ref, probs1, cnt1, cs1)):
        raw = rawp[:, g * _VP:(g + 1) * _VP]
        lg = raw * s_ref[g, :][None, :] + sb_ref[g, :][None, :]
        m = jnp.max(lg, axis=1, keepdims=True)
        e = jnp.exp(lg - m)
        p = e / jnp.sum(e, axis=1, keepdims=True)
        pa[...] += jnp.sum(p, axis=0, keepdims=True)
        k = jnp.argmax(lg, axis=1).astype(jnp.int32)
        idx_ref[...] = k
        oh = (lax.broadcasted_iota(jnp.int32, (_TB, _VP), 1)
              == k[:, None]).astype(jnp.float32)
        ca[...] += jnp.sum(oh, axis=0, keepdims=True)
        sa[...] += jnp.sum(lg, axis=0, keepdims=True)

    @pl.when(i == nblk - 1)
    def _emit():
        for r, dst in zip(scratch, acc_out):
            dst[...] = r[...]

    if last:
        @pl.when(i == nblk - 1)
        def _finish():
            invn = jnp.float32(1.0 / _N)
            pperp = jnp.float32(0.0)
            cperp = jnp.float32(0.0)
            for pa, ca in ((probs0, cnt0), (probs1, cnt1)):
                ap = pa[...] * invn
                pperp += jnp.exp(-jnp.sum(ap * jnp.log(ap + 1e-7)))
                hp = ca[...] * invn
                cperp += jnp.exp(-jnp.sum(hp * jnp.log(hp + 1e-7)))
            x0 = cs0[...] * invn
            x1 = cs1[...] * invn
            m2 = jnp.maximum(jnp.max(x0), jnp.max(x1))
            e0 = jnp.exp(x0 - m2)
            e1 = jnp.exp(x1 - m2)
            z = jnp.sum(e0) + jnp.sum(e1)
            logz = jnp.log(z)
            lent = (jnp.sum(e0 * ((x0 - m2) - logz))
                    + jnp.sum(e1 * ((x1 - m2) - logz))) / z
            lent_ref[0, 0] = lent
            cperp_ref[0, 0] = cperp
            pperp_ref[0, 0] = pperp


def _tc_stats(x2d, w, sb2, s2, accs, base, nblk, last):
    acc_sds = jax.ShapeDtypeStruct((1, _VP), jnp.float32)
    out_shape = [jax.ShapeDtypeStruct((nblk * _TB,), jnp.int32)] * 2 \
        + [acc_sds] * 6
    out_specs = [pl.BlockSpec((_TB,), lambda i: (i,))] * 2 + \
                [pl.BlockSpec((1, _VP), lambda i: (0, 0))] * 6
    in_specs = [
        pl.BlockSpec((_TB, _C), lambda i: (i + base, 0)),
        pl.BlockSpec((_G * _V, _C), lambda i: (0, 0)),
        pl.BlockSpec((_G, _VP), lambda i: (0, 0)),
        pl.BlockSpec((_G, _VP), lambda i: (0, 0)),
    ]
    if last:
        out_shape += [jax.ShapeDtypeStruct((1, 1), jnp.float32)] * 3
        out_specs += [pl.BlockSpec(memory_space=pltpu.SMEM)] * 3
        in_specs += [pl.BlockSpec((1, _VP), lambda i: (0, 0))] * 6
    return pl.pallas_call(
        functools.partial(_tc_body, nblk, last),
        grid=(nblk,),
        in_specs=in_specs,
        out_specs=out_specs,
        out_shape=out_shape,
        scratch_shapes=[pltpu.VMEM((_G * _VP, _C), jnp.float32)]
        + [pltpu.VMEM((1, _VP), jnp.float32) for _ in range(6)],
        compiler_params=pltpu.CompilerParams(
            dimension_semantics=("arbitrary",)),
    )(x2d, w, sb2, s2, *accs)


def _sc_body(base_tok, tpw, chunks, tableg_hbm, k0_hbm, k1_hbm, out_hbm,
             kv, buf0, buf1, gs0, gs1, ss0, ss1):
    wid = lax.axis_index("s") * _NC + lax.axis_index("c")
    g = wid & 1
    blk = wid >> 1
    col0 = g * _VD
    tok0 = base_tok + blk * tpw

    @pl.when(g == 0)
    def _load0():
        pltpu.sync_copy(k0_hbm.at[pl.ds(blk * tpw, tpw)], kv)

    @pl.when(g == 1)
    def _load1():
        pltpu.sync_copy(k1_hbm.at[pl.ds(blk * tpw, tpw)], kv)

    tab = tableg_hbm.at[g]
    bufs = (buf0, buf1)
    gsems = (gs0, gs1)
    ssems = (ss0, ss1)
    gh = [None, None]
    sh = [None, None]
    gh[0] = pltpu.async_copy(tab.at[kv.at[pl.ds(0, _CB)]], buf0, gs0)
    for c in range(chunks):
        n = c + 1
        if n < chunks:
            if sh[n % 2] is not None:
                sh[n % 2].wait()
            gh[n % 2] = pltpu.async_copy(
                tab.at[kv.at[pl.ds(n * _CB, _CB)]], bufs[n % 2],
                gsems[n % 2])
        gh[c % 2].wait()
        sh[c % 2] = pltpu.async_copy(
            bufs[c % 2],
            out_hbm.at[pl.ds(tok0 + c * _CB, _CB), pl.ds(col0, _VD)],
            ssems[c % 2])
    for c in range(max(0, chunks - 2), chunks):
        sh[c % 2].wait()


def _sc_gather(tableg, k0, k1, qref, base_tok, tpw):
    chunks = tpw // _CB
    mesh = plsc.VectorSubcoreMesh(core_axis_name="c", subcore_axis_name="s")
    out_type = () if qref is not None else jax.ShapeDtypeStruct(
        (_N, _G * _VD), jnp.float32)
    run = functools.partial(
        pl.kernel,
        mesh=mesh,
        out_type=out_type,
        scratch_types=[
            pltpu.VMEM((tpw,), jnp.int32),
            pltpu.VMEM((_CB, _VD), jnp.float32),
            pltpu.VMEM((_CB, _VD), jnp.float32),
            pltpu.SemaphoreType.DMA,
            pltpu.SemaphoreType.DMA,
            pltpu.SemaphoreType.DMA,
            pltpu.SemaphoreType.DMA,
        ],
    )(functools.partial(_sc_body, base_tok, tpw, chunks))
    if qref is None:
        return run(tableg, k0, k1)
    run(tableg, k0, k1, qref)


def kernel(x, W, b, vars_p, scaling):
    avg = scaling.mean()
    scale = 1.0 + 10.0 * (scaling - avg)
    s2 = jnp.pad(scale.reshape(_G, _V), ((0, 0), (0, _VP - _V)),
                 constant_values=1.0)
    sb2 = jnp.pad((b * scale).reshape(_G, _V), ((0, 0), (0, _VP - _V)),
                  constant_values=_NEG)

    x2d = x.reshape(_N, _C)
    tableg = vars_p.reshape(_G, _V, _VD)

    k0a, k1a, *accs_a = _tc_stats(x2d, W, sb2, s2, [], 0, _NBLK_A, False)
    out_a = _sc_gather(tableg, k0a, k1a, None, 0, _NSL_A // (_NW // 2))
    qref = jax.new_ref(out_a)

    out_b = _tc_stats(x2d, W, sb2, s2, accs_a, _NBLK_A, _NBLK_B, True)
    k0b, k1b = out_b[0], out_b[1]
    lent, cperp, pperp = out_b[8], out_b[9], out_b[10]
    _sc_gather(tableg, k0b, k1b, qref, _NSL_A, _NSL_B // (_NW // 2))

    q = qref[...].reshape(_B, _T, _G * _VD)
    return (q, lent.reshape(()), cperp.reshape(()), pperp.reshape(()))


# speedup vs baseline: 13.8998x; 1.0250x over previous
import functools

import jax
import jax.numpy as jnp
from jax import lax
from jax.experimental import pallas as pl
from jax.experimental.pallas import tpu as pltpu
from jax.experimental.pallas import tpu_sc as plsc

_B, _T, _C = 4, 2048, 1024
_G, _V = 2, 320
_VP = 384
_NEG = -1e30
_N = _B * _T
_VD = 256
_TB = 1024
_NBLK_A = 6
_NBLK_B = _N // _TB - _NBLK_A
_NSL_A = _NBLK_A * _TB
_NSL_B = _NBLK_B * _TB

_NC, _NS = 2, 16
_NW = _NC * _NS
_CB = 128


def _tc_body(nblk, last, *refs):
    if last:
        (x_ref, w_ref, sb_ref, s_ref, a0, a1, a2, a3, a4, a5,
         idx0_ref, idx1_ref, o0, o1, o2, o3, o4, o5,
         lent_ref, cperp_ref, pperp_ref,
         wsc, probs0, probs1, cnt0, cnt1, cs0, cs1) = refs
        acc_in = (a0, a1, a2, a3, a4, a5)
    else:
        (x_ref, w_ref, sb_ref, s_ref,
         idx0_ref, idx1_ref, o0, o1, o2, o3, o4, o5,
         wsc, probs0, probs1, cnt0, cnt1, cs0, cs1) = refs
        acc_in = None
    acc_out = (o0, o1, o2, o3, o4, o5)
    scratch = (probs0, probs1, cnt0, cnt1, cs0, cs1)
    i = pl.program_id(0)

    @pl.when(i == 0)
    def _init():
        if acc_in is None:
            for r in scratch:
                r[...] = jnp.zeros_like(r)
        else:
            for r, src in zip(scratch, acc_in):
                r[...] = src[...]
        wsc[0:_V, :] = w_ref[0:_V, :]
        wsc[_V:_VP, :] = jnp.zeros((_VP - _V, _C), jnp.float32)
        wsc[_VP:_VP + _V, :] = w_ref[_V:2 * _V, :]
        wsc[_VP + _V:2 * _VP, :] = jnp.zeros((_VP - _V, _C), jnp.float32)

    xb = x_ref[...]
    rawp = lax.dot_general(xb, wsc[...], (((1,), (1,)), ((), ())),
                           preferred_element_type=jnp.float32)
    for g, idx_ref, pa, ca, sa in (
            (0, idx0_ref, probs0, cnt0, cs0),
            (1, idx1_ref, probs1, cnt1, cs1)):
        raw = rawp[:, g * _VP:(g + 1) * _VP]
        lg = raw * s_ref[g, :][None, :] + sb_ref[g, :][None, :]
        m = jnp.max(lg, axis=1, keepdims=True)
        e = jnp.exp(lg - m)
        p = e / jnp.sum(e, axis=1, keepdims=True)
        pa[...] += jnp.sum(p, axis=0, keepdims=True)
        k = jnp.argmax(lg, axis=1).astype(jnp.int32)
        idx_ref[...] = k
        oh = (lax.broadcasted_iota(jnp.int32, (_TB, _VP), 1)
              == k[:, None]).astype(jnp.float32)
        ca[...] += jnp.sum(oh, axis=0, keepdims=True)
        sa[...] += jnp.sum(lg, axis=0, keepdims=True)

    @pl.when(i == nblk - 1)
    def _emit():
        for r, dst in zip(scratch, acc_out):
            dst[...] = r[...]

    if last:
        @pl.when(i == nblk - 1)
        def _finish():
            invn = jnp.float32(1.0 / _N)
            pperp = jnp.float32(0.0)
            cperp = jnp.float32(0.0)
            for pa, ca in ((probs0, cnt0), (probs1, cnt1)):
                ap = pa[...] * invn
                pperp += jnp.exp(-jnp.sum(ap * jnp.log(ap + 1e-7)))
                hp = ca[...] * invn
                cperp += jnp.exp(-jnp.sum(hp * jnp.log(hp + 1e-7)))
            x0 = cs0[...] * invn
            x1 = cs1[...] * invn
            m2 = jnp.maximum(jnp.max(x0), jnp.max(x1))
            e0 = jnp.exp(x0 - m2)
            e1 = jnp.exp(x1 - m2)
            z = jnp.sum(e0) + jnp.sum(e1)
            logz = jnp.log(z)
            lent = (jnp.sum(e0 * ((x0 - m2) - logz))
                    + jnp.sum(e1 * ((x1 - m2) - logz))) / z
            lent_ref[0, 0] = lent
            cperp_ref[0, 0] = cperp
            pperp_ref[0, 0] = pperp


def _tc_stats(x2d, w, sb2, s2, accs, base, nblk, last):
    acc_sds = jax.ShapeDtypeStruct((1, _VP), jnp.float32)
    out_shape = [jax.ShapeDtypeStruct((nblk * _TB,), jnp.int32)] * 2 \
        + [acc_sds] * 6
    out_specs = [pl.BlockSpec((_TB,), lambda i: (i,))] * 2 + \
                [pl.BlockSpec((1, _VP), lambda i: (0, 0))] * 6
    in_specs = [
        pl.BlockSpec((_TB, _C), lambda i: (i + base, 0)),
        pl.BlockSpec((_G * _V, _C), lambda i: (0, 0)),
        pl.BlockSpec((_G, _VP), lambda i: (0, 0)),
        pl.BlockSpec((_G, _VP), lambda i: (0, 0)),
    ]
    if last:
        out_shape += [jax.ShapeDtypeStruct((1, 1), jnp.float32)] * 3
        out_specs += [pl.BlockSpec(memory_space=pltpu.SMEM)] * 3
        in_specs += [pl.BlockSpec((1, _VP), lambda i: (0, 0))] * 6
    return pl.pallas_call(
        functools.partial(_tc_body, nblk, last),
        grid=(nblk,),
        in_specs=in_specs,
        out_specs=out_specs,
        out_shape=out_shape,
        scratch_shapes=[pltpu.VMEM((_G * _VP, _C), jnp.float32)]
        + [pltpu.VMEM((1, _VP), jnp.float32) for _ in range(6)],
        compiler_params=pltpu.CompilerParams(
            dimension_semantics=("arbitrary",)),
    )(x2d, w, sb2, s2, *accs)


def _sc_body(base_tok, tpw, chunks, tableg_hbm, k0_hbm, k1_hbm, out_hbm,
             kv, buf0, buf1, gs0, gs1, ss0, ss1):
    wid = lax.axis_index("s") * _NC + lax.axis_index("c")
    g = wid & 1
    blk = wid >> 1
    col0 = g * _VD
    tok0 = base_tok + blk * tpw

    @pl.when(g == 0)
    def _load0():
        pltpu.sync_copy(k0_hbm.at[pl.ds(blk * tpw, tpw)], kv)

    @pl.when(g == 1)
    def _load1():
        pltpu.sync_copy(k1_hbm.at[pl.ds(blk * tpw, tpw)], kv)

    tab = tableg_hbm.at[g]
    bufs = (buf0, buf1)
    gsems = (gs0, gs1)
    ssems = (ss0, ss1)
    gh = [None, None]
    sh = [None, None]
    gh[0] = pltpu.async_copy(tab.at[kv.at[pl.ds(0, _CB)]], buf0, gs0)
    for c in range(chunks):
        n = c + 1
        if n < chunks:
            if sh[n % 2] is not None:
                sh[n % 2].wait()
            gh[n % 2] = pltpu.async_copy(
                tab.at[kv.at[pl.ds(n * _CB, _CB)]], bufs[n % 2],
                gsems[n % 2])
        gh[c % 2].wait()
        sh[c % 2] = pltpu.async_copy(
            bufs[c % 2],
            out_hbm.at[pl.ds(tok0 + c * _CB, _CB), pl.ds(col0, _VD)],
            ssems[c % 2])
    for c in range(max(0, chunks - 2), chunks):
        sh[c % 2].wait()


def _sc_gather(tableg, k0, k1, qref, base_tok, tpw):
    chunks = tpw // _CB
    mesh = plsc.VectorSubcoreMesh(core_axis_name="c", subcore_axis_name="s")
    out_type = () if qref is not None else jax.ShapeDtypeStruct(
        (_N, _G * _VD), jnp.float32)
    run = functools.partial(
        pl.kernel,
        mesh=mesh,
        out_type=out_type,
        scratch_types=[
            pltpu.VMEM((tpw,), jnp.int32),
            pltpu.VMEM((_CB, _VD), jnp.float32),
            pltpu.VMEM((_CB, _VD), jnp.float32),
            pltpu.SemaphoreType.DMA,
            pltpu.SemaphoreType.DMA,
            pltpu.SemaphoreType.DMA,
            pltpu.SemaphoreType.DMA,
        ],
    )(functools.partial(_sc_body, base_tok, tpw, chunks))
    if qref is None:
        return run(tableg, k0, k1)
    run(tableg, k0, k1, qref)


def kernel(x, W, b, vars_p, scaling):
    avg = scaling.mean()
    scale = 1.0 + 10.0 * (scaling - avg)
    s2 = jnp.pad(scale.reshape(_G, _V), ((0, 0), (0, _VP - _V)),
                 constant_values=1.0)
    sb2 = jnp.pad((b * scale).reshape(_G, _V), ((0, 0), (0, _VP - _V)),
                  constant_values=_NEG)

    x2d = x.reshape(_N, _C)
    tableg = vars_p.reshape(_G, _V, _VD)

    k0a, k1a, *accs_a = _tc_stats(x2d, W, sb2, s2, [], 0, _NBLK_A, False)
    out_a = _sc_gather(tableg, k0a, k1a, None, 0, _NSL_A // (_NW // 2))
    qref = jax.new_ref(out_a)

    out_b = _tc_stats(x2d, W, sb2, s2, accs_a, _NBLK_A, _NBLK_B, True)
    k0b, k1b = out_b[0], out_b[1]
    lent, cperp, pperp = out_b[8], out_b[9], out_b[10]
    _sc_gather(tableg, k0b, k1b, qref, _NSL_A, _NSL_B // (_NW // 2))

    q = qref[...].reshape(_B, _T, _G * _VD)
    return (q, lent.reshape(()), cperp.reshape(()), pperp.reshape(()))


# speedup vs baseline: 13.9680x; 1.0049x over previous
import functools

import jax
import jax.numpy as jnp
from jax import lax
from jax.experimental import pallas as pl
from jax.experimental.pallas import tpu as pltpu
from jax.experimental.pallas import tpu_sc as plsc

_B, _T, _C = 4, 2048, 1024
_G, _V = 2, 320
_VP = 384
_NEG = -1e30
_N = _B * _T
_VD = 256
_TB = 1024
_NBLK_A = 6
_NBLK_B = _N // _TB - _NBLK_A
_NSL_A = _NBLK_A * _TB
_NSL_B = _NBLK_B * _TB

_NC, _NS = 2, 16
_NW = _NC * _NS
_CB = 128


def _tc_body(nblk, last, *refs):
    if last:
        (x_ref, w_ref, ssb_ref, a0, a1, a2, a3, a4, a5,
         idx0_ref, idx1_ref, o0, o1, o2, o3, o4, o5,
         lent_ref, cperp_ref, pperp_ref,
         wsc, ssc, sbsc, probs0, probs1, cnt0, cnt1, cs0, cs1) = refs
        acc_in = (a0, a1, a2, a3, a4, a5)
    else:
        (x_ref, w_ref, ssb_ref,
         idx0_ref, idx1_ref, o0, o1, o2, o3, o4, o5,
         wsc, ssc, sbsc, probs0, probs1, cnt0, cnt1, cs0, cs1) = refs
        acc_in = None
    acc_out = (o0, o1, o2, o3, o4, o5)
    scratch = (probs0, probs1, cnt0, cnt1, cs0, cs1)
    i = pl.program_id(0)

    @pl.when(i == 0)
    def _init():
        if acc_in is None:
            for r in scratch:
                r[...] = jnp.zeros_like(r)
        else:
            for r, src in zip(scratch, acc_in):
                r[...] = src[...]
        wsc[0:_V, :] = w_ref[0:_V, :]
        wsc[_V:_VP, :] = jnp.zeros((_VP - _V, _C), jnp.float32)
        wsc[_VP:_VP + _V, :] = w_ref[_V:2 * _V, :]
        wsc[_VP + _V:2 * _VP, :] = jnp.zeros((_VP - _V, _C), jnp.float32)
        real = lax.broadcasted_iota(jnp.int32, (_G, _VP), 1) < _V
        ssc[...] = jnp.where(real, ssb_ref[0:_G, :], 1.0)
        sbsc[...] = jnp.where(real, ssb_ref[_G:2 * _G, :], _NEG)

    xb = x_ref[...]
    rawp = lax.dot_general(xb, wsc[...], (((1,), (1,)), ((), ())),
                           preferred_element_type=jnp.float32)
    for g, idx_ref, pa, ca, sa in (
            (0, idx0_ref, probs0, cnt0, cs0),
            (1, idx1_ref, probs1, cnt1, cs1)):
        raw = rawp[:, g * _VP:(g + 1) * _VP]
        lg = raw * ssc[g, :][None, :] + sbsc[g, :][None, :]
        m = jnp.max(lg, axis=1, keepdims=True)
        e = jnp.exp(lg - m)
        p = e / jnp.sum(e, axis=1, keepdims=True)
        pa[...] += jnp.sum(p, axis=0, keepdims=True)
        k = jnp.argmax(lg, axis=1).astype(jnp.int32)
        idx_ref[...] = k
        oh = (lax.broadcasted_iota(jnp.int32, (_TB, _VP), 1)
              == k[:, None]).astype(jnp.float32)
        ca[...] += jnp.sum(oh, axis=0, keepdims=True)
        sa[...] += jnp.sum(lg, axis=0, keepdims=True)

    @pl.when(i == nblk - 1)
    def _emit():
        for r, dst in zip(scratch, acc_out):
            dst[...] = r[...]

    if last:
        @pl.when(i == nblk - 1)
        def _finish():
            invn = jnp.float32(1.0 / _N)
            pperp = jnp.float32(0.0)
            cperp = jnp.float32(0.0)
            for pa, ca in ((probs0, cnt0), (probs1, cnt1)):
                ap = pa[...] * invn
                pperp += jnp.exp(-jnp.sum(ap * jnp.log(ap + 1e-7)))
                hp = ca[...] * invn
                cperp += jnp.exp(-jnp.sum(hp * jnp.log(hp + 1e-7)))
            x0 = cs0[...] * invn
            x1 = cs1[...] * invn
            m2 = jnp.maximum(jnp.max(x0), jnp.max(x1))
            e0 = jnp.exp(x0 - m2)
            e1 = jnp.exp(x1 - m2)
            z = jnp.sum(e0) + jnp.sum(e1)
            logz = jnp.log(z)
            lent = (jnp.sum(e0 * ((x0 - m2) - logz))
                    + jnp.sum(e1 * ((x1 - m2) - logz))) / z
            lent_ref[0, 0] = lent
            cperp_ref[0, 0] = cperp
            pperp_ref[0, 0] = pperp


def _tc_stats(x2d, w, ssb, accs, base, nblk, last):
    acc_sds = jax.ShapeDtypeStruct((1, _VP), jnp.float32)
    out_shape = [jax.ShapeDtypeStruct((nblk * _TB,), jnp.int32)] * 2 \
        + [acc_sds] * 6
    out_specs = [pl.BlockSpec((_TB,), lambda i: (i,))] * 2 + \
                [pl.BlockSpec((1, _VP), lambda i: (0, 0))] * 6
    in_specs = [
        pl.BlockSpec((_TB, _C), lambda i: (i + base, 0)),
        pl.BlockSpec((_G * _V, _C), lambda i: (0, 0)),
        pl.BlockSpec((2 * _G, _VP), lambda i: (0, 0)),
    ]
    if last:
        out_shape += [jax.ShapeDtypeStruct((1, 1), jnp.float32)] * 3
        out_specs += [pl.BlockSpec(memory_space=pltpu.SMEM)] * 3
        in_specs += [pl.BlockSpec((1, _VP), lambda i: (0, 0))] * 6
    return pl.pallas_call(
        functools.partial(_tc_body, nblk, last),
        grid=(nblk,),
        in_specs=in_specs,
        out_specs=out_specs,
        out_shape=out_shape,
        scratch_shapes=[pltpu.VMEM((_G * _VP, _C), jnp.float32),
                        pltpu.VMEM((_G, _VP), jnp.float32),
                        pltpu.VMEM((_G, _VP), jnp.float32)]
        + [pltpu.VMEM((1, _VP), jnp.float32) for _ in range(6)],
        compiler_params=pltpu.CompilerParams(
            dimension_semantics=("arbitrary",)),
    )(x2d, w, ssb, *accs)


def _sc_body(base_tok, tpw, chunks, tableg_hbm, k0_hbm, k1_hbm, out_hbm,
             kv, buf0, buf1, gs0, gs1, ss0, ss1):
    wid = lax.axis_index("s") * _NC + lax.axis_index("c")
    g = wid & 1
    blk = wid >> 1
    col0 = g * _VD
    tok0 = base_tok + blk * tpw

    @pl.when(g == 0)
    def _load0():
        pltpu.sync_copy(k0_hbm.at[pl.ds(blk * tpw, tpw)], kv)

    @pl.when(g == 1)
    def _load1():
        pltpu.sync_copy(k1_hbm.at[pl.ds(blk * tpw, tpw)], kv)

    tab = tableg_hbm.at[g]
    bufs = (buf0, buf1)
    gsems = (gs0, gs1)
    ssems = (ss0, ss1)
    gh = [None, None]
    sh = [None, None]
    gh[0] = pltpu.async_copy(tab.at[kv.at[pl.ds(0, _CB)]], buf0, gs0)
    for c in range(chunks):
        n = c + 1
        if n < chunks:
            if sh[n % 2] is not None:
                sh[n % 2].wait()
            gh[n % 2] = pltpu.async_copy(
                tab.at[kv.at[pl.ds(n * _CB, _CB)]], bufs[n % 2],
                gsems[n % 2])
        gh[c % 2].wait()
        sh[c % 2] = pltpu.async_copy(
            bufs[c % 2],
            out_hbm.at[pl.ds(tok0 + c * _CB, _CB), pl.ds(col0, _VD)],
            ssems[c % 2])
    for c in range(max(0, chunks - 2), chunks):
        sh[c % 2].wait()


def _sc_gather(tableg, k0, k1, qref, base_tok, tpw):
    chunks = tpw // _CB
    mesh = plsc.VectorSubcoreMesh(core_axis_name="c", subcore_axis_name="s")
    out_type = () if qref is not None else jax.ShapeDtypeStruct(
        (_N, _G * _VD), jnp.float32)
    run = functools.partial(
        pl.kernel,
        mesh=mesh,
        out_type=out_type,
        scratch_types=[
            pltpu.VMEM((tpw,), jnp.int32),
            pltpu.VMEM((_CB, _VD), jnp.float32),
            pltpu.VMEM((_CB, _VD), jnp.float32),
            pltpu.SemaphoreType.DMA,
            pltpu.SemaphoreType.DMA,
            pltpu.SemaphoreType.DMA,
            pltpu.SemaphoreType.DMA,
        ],
    )(functools.partial(_sc_body, base_tok, tpw, chunks))
    if qref is None:
        return run(tableg, k0, k1)
    run(tableg, k0, k1, qref)


def kernel(x, W, b, vars_p, scaling):
    avg = scaling.mean()
    scale = 1.0 + 10.0 * (scaling - avg)
    ssb = jnp.pad(jnp.concatenate(
        [scale.reshape(_G, _V), (b * scale).reshape(_G, _V)]),
        ((0, 0), (0, _VP - _V)))

    x2d = x.reshape(_N, _C)
    tableg = vars_p.reshape(_G, _V, _VD)

    k0a, k1a, *accs_a = _tc_stats(x2d, W, ssb, [], 0, _NBLK_A, False)
    out_a = _sc_gather(tableg, k0a, k1a, None, 0, _NSL_A // (_NW // 2))
    qref = jax.new_ref(out_a)

    out_b = _tc_stats(x2d, W, ssb, accs_a, _NBLK_A, _NBLK_B, True)
    k0b, k1b = out_b[0], out_b[1]
    lent, cperp, pperp = out_b[8], out_b[9], out_b[10]
    _sc_gather(tableg, k0b, k1b, qref, _NSL_A, _NSL_B // (_NW // 2))

    q = qref[...].reshape(_B, _T, _G * _VD)
    return (q, lent.reshape(()), cperp.reshape(()), pperp.reshape(()))
